# baseline (speedup 1.0000x reference)
import functools

import jax
import jax.numpy as jnp
from jax import lax
from jax.experimental import pallas as pl
from jax.experimental.pallas import tpu as pltpu
from jax.experimental.pallas import tpu_sc as plsc

N = 10000
D = 128
ED = 16
K = 64

NCORES = 2
NSUB = 16
NW = NCORES * NSUB
CH = 16
NC0 = 630
NC1 = 630
NPAD = NSUB * 640
ROWS_PER_TILE = NPAD // NSUB

E_BLK = 2560


def _ceil_to(v, m):
    return (v + m - 1) // m * m


def _edge_mm_body(ea, w, b, o):
    o[...] = (jnp.dot(ea[...].astype(jnp.bfloat16),
                      w[...].astype(jnp.bfloat16),
                      preferred_element_type=jnp.float32)
              + b[...])


def _edge_mm(ea, We, be, ep):
    e_total = ea.shape[0]
    grid = (e_total // E_BLK,)
    return pl.pallas_call(
        _edge_mm_body,
        grid=grid,
        in_specs=[
            pl.BlockSpec((E_BLK, ED), lambda i: (i, 0)),
            pl.BlockSpec((ED, D), lambda i: (0, 0)),
            pl.BlockSpec((1, D), lambda i: (0, 0)),
        ],
        out_specs=pl.BlockSpec((E_BLK, D), lambda i: (i, 0)),
        out_shape=jax.ShapeDtypeStruct((ep, D), jnp.float32),
    )(ea, We, be.reshape(1, D))


def _edge_pass_body(e_hbm, src_hbm, dst_hbm, tab_hbm, out_hbm,
                    sbuf, dbuf, eb0, eb1, xb0, xb1, mb0, mb1, aggsh,
                    es0, es1, gs0, gs1, ss0, ss1):
    cid = lax.axis_index("c")
    sid = lax.axis_index("s")
    ebufs, xbufs, mbufs = (eb0, eb1), (xb0, xb1), (mb0, mb1)
    esem, gsem, ssem = (es0, es1), (gs0, gs1), (ss0, ss1)
    nch = jnp.where(cid == 0, NC0, NC1)
    cb0 = jnp.where(cid == 0, sid * NC0, NSUB * NC0 + sid * NC1)

    def zrow(r, _):
        for cc in range(8):
            mb0[r, pl.ds(cc * 16, 16)] = jnp.zeros((16,), jnp.float32)
        return 0
    lax.fori_loop(0, CH, zrow, 0)

    def zcp(i, _):
        pltpu.sync_copy(mb0.at[pl.ds(0, 16), :],
                        aggsh.at[pl.ds(sid * ROWS_PER_TILE + i * 16, 16), :])
        return 0
    lax.fori_loop(0, ROWS_PER_TILE // 16, zcp, 0)
    plsc.subcore_barrier()

    npe = NC1 * CH
    pltpu.sync_copy(src_hbm.at[pl.ds(cb0 * CH, npe)], sbuf)
    pltpu.sync_copy(dst_hbm.at[pl.ds(cb0 * CH, npe)], dbuf)

    erow0 = cb0 * CH

    def start_e(i, b):
        pltpu.async_copy(e_hbm.at[pl.ds(erow0 + i * CH, CH), :],
                         ebufs[b], esem[b])

    def start_g(i, b):
        pltpu.async_copy(tab_hbm.at[sbuf.at[pl.ds(i * CH, CH)]],
                         xbufs[b], gsem[b])

    start_e(0, 0)
    start_g(0, 0)
    start_e(1, 1)
    start_g(1, 1)

    def pair(g, _):
        for b in range(2):
            i = 2 * g + b
            pltpu.make_async_copy(e_hbm.at[pl.ds(erow0 + i * CH, CH), :],
                                  ebufs[b], esem[b]).wait()
            pltpu.make_async_copy(tab_hbm.at[sbuf.at[pl.ds(i * CH, CH)]],
                                  xbufs[b], gsem[b]).wait()

            @pl.when(g >= 1)
            def _():
                pltpu.make_async_copy(
                    mbufs[b], aggsh.at[dbuf.at[pl.ds(i * CH, CH)]],
                    ssem[b]).wait()

            def rw(r, _):
                for rr in range(2):
                    for cc in range(8):
                        s = pl.ds(cc * 16, 16)
                        mbufs[b][2 * r + rr, s] = jnp.maximum(
                            ebufs[b][2 * r + rr, s] + xbufs[b][2 * r + rr, s],
                            0.0)
                return 0
            lax.fori_loop(0, CH // 2, rw, 0)

            pltpu.async_copy(mbufs[b],
                             aggsh.at[dbuf.at[pl.ds(i * CH, CH)]],
                             ssem[b], add=True)

            @pl.when(g < nch // 2 - 1)
            def _():
                start_e(i + 2, b)
                start_g(i + 2, b)
        return 0
    lax.fori_loop(0, nch // 2, pair, 0)

    for b in range(2):
        i = nch - 2 + b
        pltpu.make_async_copy(mbufs[b],
                              aggsh.at[dbuf.at[pl.ds(i * CH, CH)]],
                              ssem[b]).wait()

    plsc.subcore_barrier()
    r0 = sid * ROWS_PER_TILE
    pltpu.sync_copy(aggsh.at[pl.ds(r0, ROWS_PER_TILE), :],
                    out_hbm.at[cid, pl.ds(r0, ROWS_PER_TILE), :])


def _edge_pass(e, srcp, dstp, table, ep):
    mesh = plsc.VectorSubcoreMesh(core_axis_name="c", subcore_axis_name="s")
    return pl.kernel(
        _edge_pass_body,
        out_type=jax.ShapeDtypeStruct((NCORES, NPAD, D), jnp.float32),
        mesh=mesh,
        scratch_types=[
            pltpu.VMEM((NC1 * CH,), jnp.int32),
            pltpu.VMEM((NC1 * CH,), jnp.int32),
            pltpu.VMEM((CH, D), jnp.float32),
            pltpu.VMEM((CH, D), jnp.float32),
            pltpu.VMEM((CH, D), jnp.float32),
            pltpu.VMEM((CH, D), jnp.float32),
            pltpu.VMEM((CH, D), jnp.float32),
            pltpu.VMEM((CH, D), jnp.float32),
            pltpu.VMEM_SHARED((NPAD, D), jnp.float32),
            pltpu.SemaphoreType.DMA,
            pltpu.SemaphoreType.DMA,
            pltpu.SemaphoreType.DMA,
            pltpu.SemaphoreType.DMA,
            pltpu.SemaphoreType.DMA,
            pltpu.SemaphoreType.DMA,
        ],
    )(e, srcp, dstp, table)


def _node_mlp_body(x, a0, a1, w1, b1, w2, b2, out):
    h = x[...] + a0[0] + a1[0]
    t = jnp.maximum(jnp.dot(h, w1[...], preferred_element_type=jnp.float32)
                    + b1[...], 0.0)
    h1 = jnp.dot(t, w2[...], preferred_element_type=jnp.float32) + b2[...]
    out[...] = jnp.maximum(h1, 0.0)


def _node_mlp(x, agg, W1, b1, W2, b2):
    blk = 400
    grid = (N // blk,)
    return pl.pallas_call(
        _node_mlp_body,
        grid=grid,
        in_specs=[
            pl.BlockSpec((blk, D), lambda i: (i, 0)),
            pl.BlockSpec((1, blk, D), lambda i: (0, i, 0)),
            pl.BlockSpec((1, blk, D), lambda i: (1, i, 0)),
            pl.BlockSpec((D, D), lambda i: (0, 0)),
            pl.BlockSpec((1, D), lambda i: (0, 0)),
            pl.BlockSpec((D, D), lambda i: (0, 0)),
            pl.BlockSpec((1, D), lambda i: (0, 0)),
        ],
        out_specs=pl.BlockSpec((blk, D), lambda i: (i, 0)),
        out_shape=jax.ShapeDtypeStruct((N, D), jnp.float32),
    )(x, agg, agg, W1, b1.reshape(1, D), W2, b2.reshape(1, D))


def _head_body(hr, a0, a1, w21, b21, w22, b22,
               l1a, l1b_, l1c, l1bias, l2w, l2b, out):
    h2 = hr[...] + a0[0] + a1[0]
    t = jnp.maximum(jnp.dot(h2, w21[...], preferred_element_type=jnp.float32)
                    + b21[...], 0.0)
    emb = jnp.dot(t, w22[...], preferred_element_type=jnp.float32) + b22[...]
    curr = emb[0:1, :]
    dest = emb[1:2, :]
    nbr = emb[8:72, :]
    hq = (jnp.dot(curr, l1a[...], preferred_element_type=jnp.float32)
          + jnp.dot(dest, l1b_[...], preferred_element_type=jnp.float32)
          + jnp.dot(nbr, l1c[...], preferred_element_type=jnp.float32)
          + l1bias[...])
    hq = jnp.maximum(hq, 0.0)
    q = jnp.sum(hq * l2w[...], axis=1, keepdims=True) + l2b[...]
    out[...] = jnp.broadcast_to(q, (K, D))


def _head(hr72, agg72, W21, b21, W22, b22, L1W, L1b, L2W, L2b):
    l1a = L1W[0:D, :]
    l1b_ = L1W[D:2 * D, :]
    l1c = L1W[2 * D:3 * D, :]
    full = lambda shape: pl.BlockSpec(shape, lambda: tuple(0 for _ in shape))
    return pl.pallas_call(
        _head_body,
        in_specs=[
            full((72, D)),
            full((1, 72, D)),
            full((1, 72, D)),
            full((D, D)), full((1, D)),
            full((D, D)), full((1, D)),
            full((D, D)), full((D, D)), full((D, D)), full((1, D)),
            full((1, D)), full((1, 1)),
        ],
        out_specs=full((K, D)),
        out_shape=jax.ShapeDtypeStruct((K, D), jnp.float32),
    )(hr72, agg72[0:1], agg72[1:2], W21, b21.reshape(1, D),
      W22, b22.reshape(1, D), l1a, l1b_, l1c, L1b.reshape(1, D),
      L2W.reshape(1, D), L2b.reshape(1, 1))


def kernel(x, edge_index, curr_idx, dest_idx, neighbor_indices, edge_attr,
           We1, be1, W11, b11, W12, b12,
           We2, be2, W21, b21, W22, b22,
           L1W, L1b, L2W, L2b):
    e_total = edge_index.shape[1]
    ep = NSUB * (NC0 + NC1) * CH
    npad_e = ep - e_total

    src = edge_index[0]
    dst = edge_index[1]
    rows = e_total // D
    prow = npad_e // D
    srcp = jnp.pad(src.reshape(rows, D), ((0, prow), (0, 0))).reshape(ep)
    dstp = jnp.pad(dst.reshape(rows, D), ((0, prow), (0, 0)),
                   constant_values=N).reshape(ep)

    e1 = _edge_mm(edge_attr, We1, be1, ep)
    e2 = _edge_mm(edge_attr, We2, be2, ep)

    agg1 = _edge_pass(e1, srcp, dstp, x, ep)
    hr = _node_mlp(x, agg1, W11, b11, W12, b12)

    agg2 = _edge_pass(e2, srcp, dstp, hr, ep)

    ci = jnp.reshape(jnp.asarray(curr_idx, jnp.int32), (1,))
    di = jnp.reshape(jnp.asarray(dest_idx, jnp.int32), (1,))
    idx72 = jnp.concatenate([ci, di, jnp.zeros((6,), jnp.int32),
                             neighbor_indices.astype(jnp.int32)])
    hr72 = hr[idx72]
    agg72 = agg2[:, idx72, :]

    qmat = _head(hr72, agg72, W21, b21, W22, b22, L1W, L1b, L2W, L2b)
    return qmat[:, 0]


# speedup vs baseline: 1.0772x; 1.0772x over previous
import functools

import jax
import jax.numpy as jnp
from jax import lax
from jax.experimental import pallas as pl
from jax.experimental.pallas import tpu as pltpu
from jax.experimental.pallas import tpu_sc as plsc

N = 10000
D = 128
ED = 16
K = 64

NCORES = 2
NSUB = 16
NW = NCORES * NSUB
CH = 24
NC0 = 420
NC1 = 420
NPAD = NSUB * 640
ROWS_PER_TILE = NPAD // NSUB

E_BLK = 2560


def _ceil_to(v, m):
    return (v + m - 1) // m * m


def _edge_mm_body(ea, w, b, o):
    o[...] = (jnp.dot(ea[...].astype(jnp.bfloat16),
                      w[...].astype(jnp.bfloat16),
                      preferred_element_type=jnp.float32)
              + b[...])


def _edge_mm(ea, We, be, ep):
    e_total = ea.shape[0]
    grid = (e_total // E_BLK,)
    return pl.pallas_call(
        _edge_mm_body,
        grid=grid,
        in_specs=[
            pl.BlockSpec((E_BLK, ED), lambda i: (i, 0)),
            pl.BlockSpec((ED, D), lambda i: (0, 0)),
            pl.BlockSpec((1, D), lambda i: (0, 0)),
        ],
        out_specs=pl.BlockSpec((E_BLK, D), lambda i: (i, 0)),
        out_shape=jax.ShapeDtypeStruct((ep, D), jnp.float32),
    )(ea, We, be.reshape(1, D))


def _edge_pass_body(e_hbm, src_hbm, dst_hbm, tab_hbm, out_hbm,
                    sbuf, dbuf, eb0, eb1, xb0, xb1, mb0, mb1, aggsh,
                    es0, es1, gs0, gs1, ss0, ss1):
    cid = lax.axis_index("c")
    sid = lax.axis_index("s")
    ebufs, xbufs, mbufs = (eb0, eb1), (xb0, xb1), (mb0, mb1)
    esem, gsem, ssem = (es0, es1), (gs0, gs1), (ss0, ss1)
    nch = jnp.where(cid == 0, NC0, NC1)
    cb0 = jnp.where(cid == 0, sid * NC0, NSUB * NC0 + sid * NC1)

    def zrow(r, _):
        for cc in range(8):
            mb0[r, pl.ds(cc * 16, 16)] = jnp.zeros((16,), jnp.float32)
        return 0
    lax.fori_loop(0, CH, zrow, 0)

    def zcp(i, _):
        pltpu.sync_copy(mb0.at[pl.ds(0, 16), :],
                        aggsh.at[pl.ds(sid * ROWS_PER_TILE + i * 16, 16), :])
        return 0
    lax.fori_loop(0, ROWS_PER_TILE // 16, zcp, 0)
    plsc.subcore_barrier()

    npe = NC1 * CH
    pltpu.sync_copy(src_hbm.at[pl.ds(cb0 * CH, npe)], sbuf)
    pltpu.sync_copy(dst_hbm.at[pl.ds(cb0 * CH, npe)], dbuf)

    erow0 = cb0 * CH

    def start_e(i, b):
        pltpu.async_copy(e_hbm.at[pl.ds(erow0 + i * CH, CH), :],
                         ebufs[b], esem[b])

    def start_g(i, b):
        pltpu.async_copy(tab_hbm.at[sbuf.at[pl.ds(i * CH, CH)]],
                         xbufs[b], gsem[b])

    start_e(0, 0)
    start_g(0, 0)
    start_e(1, 1)
    start_g(1, 1)

    def pair(g, _):
        for b in range(2):
            i = 2 * g + b
            pltpu.make_async_copy(e_hbm.at[pl.ds(erow0 + i * CH, CH), :],
                                  ebufs[b], esem[b]).wait()
            pltpu.make_async_copy(tab_hbm.at[sbuf.at[pl.ds(i * CH, CH)]],
                                  xbufs[b], gsem[b]).wait()

            @pl.when(g >= 1)
            def _():
                pltpu.make_async_copy(
                    mbufs[b], aggsh.at[dbuf.at[pl.ds(i * CH, CH)]],
                    ssem[b]).wait()

            def rw(r, _):
                for rr in range(3):
                    for cc in range(8):
                        s = pl.ds(cc * 16, 16)
                        mbufs[b][3 * r + rr, s] = jnp.maximum(
                            ebufs[b][3 * r + rr, s] + xbufs[b][3 * r + rr, s],
                            0.0)
                return 0
            lax.fori_loop(0, CH // 3, rw, 0)

            pltpu.async_copy(mbufs[b],
                             aggsh.at[dbuf.at[pl.ds(i * CH, CH)]],
                             ssem[b], add=True)

            @pl.when(g < nch // 2 - 1)
            def _():
                start_e(i + 2, b)
                start_g(i + 2, b)
        return 0
    lax.fori_loop(0, nch // 2, pair, 0)

    for b in range(2):
        i = nch - 2 + b
        pltpu.make_async_copy(mbufs[b],
                              aggsh.at[dbuf.at[pl.ds(i * CH, CH)]],
                              ssem[b]).wait()

    plsc.subcore_barrier()
    r0 = sid * ROWS_PER_TILE
    pltpu.sync_copy(aggsh.at[pl.ds(r0, ROWS_PER_TILE), :],
                    out_hbm.at[cid, pl.ds(r0, ROWS_PER_TILE), :])


def _edge_pass(e, srcp, dstp, table, ep):
    mesh = plsc.VectorSubcoreMesh(core_axis_name="c", subcore_axis_name="s")
    return pl.kernel(
        _edge_pass_body,
        out_type=jax.ShapeDtypeStruct((NCORES, NPAD, D), jnp.float32),
        mesh=mesh,
        scratch_types=[
            pltpu.VMEM((NC1 * CH,), jnp.int32),
            pltpu.VMEM((NC1 * CH,), jnp.int32),
            pltpu.VMEM((CH, D), jnp.float32),
            pltpu.VMEM((CH, D), jnp.float32),
            pltpu.VMEM((CH, D), jnp.float32),
            pltpu.VMEM((CH, D), jnp.float32),
            pltpu.VMEM((CH, D), jnp.float32),
            pltpu.VMEM((CH, D), jnp.float32),
            pltpu.VMEM_SHARED((NPAD, D), jnp.float32),
            pltpu.SemaphoreType.DMA,
            pltpu.SemaphoreType.DMA,
            pltpu.SemaphoreType.DMA,
            pltpu.SemaphoreType.DMA,
            pltpu.SemaphoreType.DMA,
            pltpu.SemaphoreType.DMA,
        ],
    )(e, srcp, dstp, table)


def _node_mlp_body(x, a0, a1, w1, b1, w2, b2, out):
    h = x[...] + a0[0] + a1[0]
    t = jnp.maximum(jnp.dot(h, w1[...], preferred_element_type=jnp.float32)
                    + b1[...], 0.0)
    h1 = jnp.dot(t, w2[...], preferred_element_type=jnp.float32) + b2[...]
    out[...] = jnp.maximum(h1, 0.0)


def _node_mlp(x, agg, W1, b1, W2, b2):
    blk = 400
    grid = (N // blk,)
    return pl.pallas_call(
        _node_mlp_body,
        grid=grid,
        in_specs=[
            pl.BlockSpec((blk, D), lambda i: (i, 0)),
            pl.BlockSpec((1, blk, D), lambda i: (0, i, 0)),
            pl.BlockSpec((1, blk, D), lambda i: (1, i, 0)),
            pl.BlockSpec((D, D), lambda i: (0, 0)),
            pl.BlockSpec((1, D), lambda i: (0, 0)),
            pl.BlockSpec((D, D), lambda i: (0, 0)),
            pl.BlockSpec((1, D), lambda i: (0, 0)),
        ],
        out_specs=pl.BlockSpec((blk, D), lambda i: (i, 0)),
        out_shape=jax.ShapeDtypeStruct((N, D), jnp.float32),
    )(x, agg, agg, W1, b1.reshape(1, D), W2, b2.reshape(1, D))


def _head_body(hr, a0, a1, w21, b21, w22, b22,
               l1a, l1b_, l1c, l1bias, l2w, l2b, out):
    h2 = hr[...] + a0[0] + a1[0]
    t = jnp.maximum(jnp.dot(h2, w21[...], preferred_element_type=jnp.float32)
                    + b21[...], 0.0)
    emb = jnp.dot(t, w22[...], preferred_element_type=jnp.float32) + b22[...]
    curr = emb[0:1, :]
    dest = emb[1:2, :]
    nbr = emb[8:72, :]
    hq = (jnp.dot(curr, l1a[...], preferred_element_type=jnp.float32)
          + jnp.dot(dest, l1b_[...], preferred_element_type=jnp.float32)
          + jnp.dot(nbr, l1c[...], preferred_element_type=jnp.float32)
          + l1bias[...])
    hq = jnp.maximum(hq, 0.0)
    q = jnp.sum(hq * l2w[...], axis=1, keepdims=True) + l2b[...]
    out[...] = jnp.broadcast_to(q, (K, D))


def _head(hr72, agg72, W21, b21, W22, b22, L1W, L1b, L2W, L2b):
    l1a = L1W[0:D, :]
    l1b_ = L1W[D:2 * D, :]
    l1c = L1W[2 * D:3 * D, :]
    full = lambda shape: pl.BlockSpec(shape, lambda: tuple(0 for _ in shape))
    return pl.pallas_call(
        _head_body,
        in_specs=[
            full((72, D)),
            full((1, 72, D)),
            full((1, 72, D)),
            full((D, D)), full((1, D)),
            full((D, D)), full((1, D)),
            full((D, D)), full((D, D)), full((D, D)), full((1, D)),
            full((1, D)), full((1, 1)),
        ],
        out_specs=full((K, D)),
        out_shape=jax.ShapeDtypeStruct((K, D), jnp.float32),
    )(hr72, agg72[0:1], agg72[1:2], W21, b21.reshape(1, D),
      W22, b22.reshape(1, D), l1a, l1b_, l1c, L1b.reshape(1, D),
      L2W.reshape(1, D), L2b.reshape(1, 1))


def kernel(x, edge_index, curr_idx, dest_idx, neighbor_indices, edge_attr,
           We1, be1, W11, b11, W12, b12,
           We2, be2, W21, b21, W22, b22,
           L1W, L1b, L2W, L2b):
    e_total = edge_index.shape[1]
    ep = NSUB * (NC0 + NC1) * CH
    npad_e = ep - e_total

    src = edge_index[0]
    dst = edge_index[1]
    rows = e_total // D
    prow = npad_e // D
    srcp = jnp.pad(src.reshape(rows, D), ((0, prow), (0, 0))).reshape(ep)
    dstp = jnp.pad(dst.reshape(rows, D), ((0, prow), (0, 0)),
                   constant_values=N).reshape(ep)

    e1 = _edge_mm(edge_attr, We1, be1, ep)
    e2 = _edge_mm(edge_attr, We2, be2, ep)

    agg1 = _edge_pass(e1, srcp, dstp, x, ep)
    hr = _node_mlp(x, agg1, W11, b11, W12, b12)

    agg2 = _edge_pass(e2, srcp, dstp, hr, ep)

    ci = jnp.reshape(jnp.asarray(curr_idx, jnp.int32), (1,))
    di = jnp.reshape(jnp.asarray(dest_idx, jnp.int32), (1,))
    idx72 = jnp.concatenate([ci, di, jnp.zeros((6,), jnp.int32),
                             neighbor_indices.astype(jnp.int32)])
    hr72 = hr[idx72]
    agg72 = agg2[:, idx72, :]

    qmat = _head(hr72, agg72, W21, b21, W22, b22, L1W, L1b, L2W, L2b)
    return qmat[:, 0]
